# Initial kernel scaffold; baseline (speedup 1.0000x reference)
import functools

import jax
import jax.numpy as jnp
from jax import lax
from jax.experimental import pallas as pl
from jax.experimental.pallas import tpu as pltpu
from jax.experimental.pallas import tpu_sc as plsc

N_EDGES_C = 6_400_000
NSEG = 1000
NSEG_PAD = 1024
NC = 2
NS = 16
NLANE = 16
NW = NC * NS
EDGES_PER_W = N_EDGES_C // NW
CHUNK = 20_000
NCHUNK = EDGES_PER_W // CHUNK
VPC = CHUNK // NLANE


def _sc_body(x_hbm, idx_hbm, out_hbm, xbuf, ibuf, acc_s, acc_c, rpart,
             idxlist, shared):
    cid = lax.axis_index("c")
    sid = lax.axis_index("s")
    wid = cid * NS + sid

    lane = lax.iota(jnp.int32, NLANE)
    ones = jnp.full((NLANE,), 1.0, jnp.float32)
    zeros = jnp.zeros((NLANE,), jnp.float32)

    def zero_body(i, _):
        acc_s[pl.ds(i * NLANE, NLANE)] = zeros
        acc_c[pl.ds(i * NLANE, NLANE)] = zeros
        return 0

    lax.fori_loop(0, NSEG_PAD, zero_body, 0)

    def idx_body(j, _):
        idxlist[pl.ds(j * NLANE, NLANE)] = lane + j * NLANE
        return 0

    lax.fori_loop(0, 128 // NLANE, idx_body, 0)

    base = wid * EDGES_PER_W

    for ch in range(NCHUNK):
        off = base + ch * CHUNK
        pltpu.sync_copy(x_hbm.at[pl.ds(off, CHUNK)], xbuf)
        pltpu.sync_copy(idx_hbm.at[pl.ds(off, CHUNK)], ibuf)

        def chunk_body(i, _):
            o = i * NLANE
            iv = ibuf[pl.ds(o, NLANE)]
            xv = xbuf[pl.ds(o, NLANE)]
            addr = iv * NLANE + lane
            plsc.addupdate_scatter(acc_s, [addr], xv)
            plsc.addupdate_scatter(acc_c, [addr], ones)
            return 0

        lax.fori_loop(0, VPC, chunk_body, 0)

    m15 = lane == 15

    def red_body(r, _):
        srow = acc_s[pl.ds(r * NLANE, NLANE)]
        crow = acc_c[pl.ds(r * NLANE, NLANE)]
        cs = plsc.cumsum(srow)
        cc = plsc.cumsum(crow)
        rowv = jnp.full((NLANE,), r >> 4, jnp.int32)
        colv = jnp.full((NLANE,), r & 15, jnp.int32)
        plsc.store_scatter(rpart, [rowv, colv], cs, mask=m15)
        plsc.store_scatter(rpart, [rowv + 64, colv], cc, mask=m15)
        return 0

    lax.fori_loop(0, NSEG_PAD, red_body, 0)

    @pl.when(sid == 0)
    def _():
        pltpu.sync_copy(rpart, shared)

    plsc.subcore_barrier()

    @pl.when(sid != 0)
    def _():
        pltpu.sync_copy(rpart, shared.at[idxlist], add=True)

    plsc.subcore_barrier()

    @pl.when(sid == 0)
    def _():
        pltpu.sync_copy(shared, out_hbm.at[cid])


_segment_partials = functools.partial(
    pl.kernel,
    out_type=jax.ShapeDtypeStruct((NC, 128, NLANE), jnp.float32),
    mesh=plsc.VectorSubcoreMesh(core_axis_name="c", subcore_axis_name="s"),
    scratch_types=[
        pltpu.VMEM((CHUNK,), jnp.float32),
        pltpu.VMEM((CHUNK,), jnp.int32),
        pltpu.VMEM((NSEG_PAD * NLANE,), jnp.float32),
        pltpu.VMEM((NSEG_PAD * NLANE,), jnp.float32),
        pltpu.VMEM((128, NLANE), jnp.float32),
        pltpu.VMEM((128,), jnp.int32),
        pltpu.VMEM_SHARED((128, NLANE), jnp.float32),
    ],
)(_sc_body)


def _finalize_body(s_ref, c_ref, t_ref, o_ref):
    s = s_ref[0] + s_ref[1]
    c = c_ref[0] + c_ref[1]
    seg = (lax.broadcasted_iota(jnp.int32, (8, 128), 0) * 128
           + lax.broadcasted_iota(jnp.int32, (8, 128), 1))
    mean = s / c
    d = jnp.where(seg < NSEG, mean - t_ref[...], 0.0)
    loss = jnp.sum(d * d) * jnp.float32(0.01 / NSEG)
    o_ref[...] = jnp.full((8, 128), loss, jnp.float32)


def kernel(x, idx, target_mean_weights):
    partials = _segment_partials(x, idx)
    p = partials.reshape(NC, 2, NSEG_PAD)
    sums = p[:, 0].reshape(NC, 8, 128)
    counts = p[:, 1].reshape(NC, 8, 128)
    t = jnp.pad(target_mean_weights, (0, NSEG_PAD - NSEG)).reshape(8, 128)
    out = pl.pallas_call(
        _finalize_body,
        out_shape=jax.ShapeDtypeStruct((8, 128), jnp.float32),
    )(sums, counts, t)
    return out[0, 0]


# speedup vs baseline: 78.2509x; 78.2509x over previous
import functools

import jax
import jax.numpy as jnp
from jax import lax
from jax.experimental import pallas as pl
from jax.experimental.pallas import tpu as pltpu
from jax.experimental.pallas import tpu_sc as plsc

N_EDGES_C = 6_400_000
NSEG = 1000
NSEG_PAD = 1024
NC = 2
NS = 16
NLANE = 16
NW = NC * NS
EDGES_PER_W = N_EDGES_C // NW
CHUNK = 20_000
NCHUNK = EDGES_PER_W // CHUNK
VPC = CHUNK // NLANE


def _sc_body(x_hbm, idx_hbm, out_hbm, xbuf, ibuf, acc_s, acc_c, rpart,
             tbuf, shared):
    cid = lax.axis_index("c")
    sid = lax.axis_index("s")
    wid = cid * NS + sid

    lane = lax.iota(jnp.int32, NLANE)
    ones = jnp.full((NLANE,), 1.0, jnp.float32)
    zeros = jnp.zeros((NLANE,), jnp.float32)

    def zero_body(i, _):
        acc_s[pl.ds(i * NLANE, NLANE)] = zeros
        acc_c[pl.ds(i * NLANE, NLANE)] = zeros
        return 0

    lax.fori_loop(0, NSEG_PAD, zero_body, 0)

    base = wid * EDGES_PER_W

    for ch in range(NCHUNK):
        off = base + ch * CHUNK
        pltpu.sync_copy(x_hbm.at[pl.ds(off, CHUNK)], xbuf)
        pltpu.sync_copy(idx_hbm.at[pl.ds(off, CHUNK)], ibuf)

        def chunk_body(i, _):
            o = i * NLANE
            iv = ibuf[pl.ds(o, NLANE)]
            xv = xbuf[pl.ds(o, NLANE)]
            addr = iv * NLANE + lane
            plsc.addupdate_scatter(acc_s, [addr], xv)
            plsc.addupdate_scatter(acc_c, [addr], ones)
            return 0

        lax.fori_loop(0, VPC, chunk_body, 0)

    m15 = lane == 15

    def red_body(r, _):
        srow = acc_s[pl.ds(r * NLANE, NLANE)]
        crow = acc_c[pl.ds(r * NLANE, NLANE)]
        cs = plsc.cumsum(srow)
        cc = plsc.cumsum(crow)
        addr = jnp.full((NLANE,), r, jnp.int32)
        plsc.store_scatter(rpart, [addr], cs, mask=m15)
        plsc.store_scatter(rpart, [addr + NSEG_PAD], cc, mask=m15)
        return 0

    lax.fori_loop(0, NSEG_PAD, red_body, 0)

    pltpu.sync_copy(rpart, shared.at[sid])
    plsc.subcore_barrier()

    @pl.when(sid == 0)
    def _():
        for t in range(1, NS):
            pltpu.sync_copy(shared.at[t], tbuf)

            def add_body(j, _):
                sl = pl.ds(j * NLANE, NLANE)
                rpart[sl] = rpart[sl] + tbuf[sl]
                return 0

            lax.fori_loop(0, 2 * NSEG_PAD // NLANE, add_body, 0)
        pltpu.sync_copy(rpart, out_hbm.at[cid])


_segment_partials = functools.partial(
    pl.kernel,
    out_type=jax.ShapeDtypeStruct((NC, 2 * NSEG_PAD), jnp.float32),
    mesh=plsc.VectorSubcoreMesh(core_axis_name="c", subcore_axis_name="s"),
    compiler_params=pltpu.CompilerParams(needs_layout_passes=False),
    scratch_types=[
        pltpu.VMEM((CHUNK,), jnp.float32),
        pltpu.VMEM((CHUNK,), jnp.int32),
        pltpu.VMEM((NSEG_PAD * NLANE,), jnp.float32),
        pltpu.VMEM((NSEG_PAD * NLANE,), jnp.float32),
        pltpu.VMEM((2 * NSEG_PAD,), jnp.float32),
        pltpu.VMEM((2 * NSEG_PAD,), jnp.float32),
        pltpu.VMEM_SHARED((NS, 2 * NSEG_PAD), jnp.float32),
    ],
)(_sc_body)


def _finalize_body(s_ref, c_ref, t_ref, o_ref):
    s = s_ref[0] + s_ref[1]
    c = c_ref[0] + c_ref[1]
    seg = (lax.broadcasted_iota(jnp.int32, (8, 128), 0) * 128
           + lax.broadcasted_iota(jnp.int32, (8, 128), 1))
    mean = s / c
    d = jnp.where(seg < NSEG, mean - t_ref[...], 0.0)
    loss = jnp.sum(d * d) * jnp.float32(0.01 / NSEG)
    o_ref[...] = jnp.full((8, 128), loss, jnp.float32)


def kernel(x, idx, target_mean_weights):
    partials = _segment_partials(x, idx)
    p = partials.reshape(NC, 2, NSEG_PAD)
    sums = p[:, 0].reshape(NC, 8, 128)
    counts = p[:, 1].reshape(NC, 8, 128)
    t = jnp.pad(target_mean_weights, (0, NSEG_PAD - NSEG)).reshape(8, 128)
    out = pl.pallas_call(
        _finalize_body,
        out_shape=jax.ShapeDtypeStruct((8, 128), jnp.float32),
    )(sums, counts, t)
    return out[0, 0]


# speedup vs baseline: 188.5168x; 2.4091x over previous
import functools

import jax
import jax.numpy as jnp
from jax import lax
from jax.experimental import pallas as pl
from jax.experimental.pallas import tpu as pltpu
from jax.experimental.pallas import tpu_sc as plsc

N_EDGES_C = 6_400_000
NSEG = 1000
NSEG_PAD = 1024
NC = 2
NS = 16
NLANE = 16
NW = NC * NS
EDGES_PER_W = N_EDGES_C // NW
CHUNK = 20_000
NCHUNK = EDGES_PER_W // CHUNK
VPC = CHUNK // NLANE


def _sc_body(x_hbm, idx_hbm, out_hbm, xbuf0, xbuf1, ibuf0, ibuf1, acc_s,
             acc_c, rpart, tbuf, shared, semx0, semx1, semi0, semi1):
    cid = lax.axis_index("c")
    sid = lax.axis_index("s")
    wid = cid * NS + sid

    lane = lax.iota(jnp.int32, NLANE)
    ones = jnp.full((NLANE,), 1.0, jnp.float32)
    zeros = jnp.zeros((NLANE,), jnp.float32)
    semx = [semx0, semx1]
    semi = [semi0, semi1]
    xbufs = [xbuf0, xbuf1]
    ibufs = [ibuf0, ibuf1]

    base = wid * EDGES_PER_W

    def issue(ch, b):
        off = base + ch * CHUNK
        hx = pltpu.async_copy(x_hbm.at[pl.ds(off, CHUNK)], xbufs[b], semx[b])
        hi = pltpu.async_copy(idx_hbm.at[pl.ds(off, CHUNK)], ibufs[b],
                              semi[b])
        return hx, hi

    pending = [issue(0, 0), None]

    @plsc.parallel_loop(0, NSEG_PAD, unroll=8)
    def _(i):
        acc_s[pl.ds(i * NLANE, NLANE)] = zeros
        acc_c[pl.ds(i * NLANE, NLANE)] = zeros

    for ch in range(NCHUNK):
        b = ch & 1
        hx, hi = pending[b]
        hx.wait()
        hi.wait()
        if ch + 1 < NCHUNK:
            pending[1 - b] = issue(ch + 1, 1 - b)
        ib = ibufs[b]
        xb = xbufs[b]

        @plsc.parallel_loop(0, VPC, unroll=8)
        def _(i):
            o = i * NLANE
            iv = ib[pl.ds(o, NLANE)]
            xv = xb[pl.ds(o, NLANE)]
            addr = iv * NLANE + lane
            plsc.addupdate_scatter(acc_s, [addr], xv)
            plsc.addupdate_scatter(acc_c, [addr], ones)

    m15 = lane == 15

    @plsc.parallel_loop(0, NSEG_PAD, unroll=4)
    def _(r):
        srow = acc_s[pl.ds(r * NLANE, NLANE)]
        crow = acc_c[pl.ds(r * NLANE, NLANE)]
        cs = plsc.cumsum(srow)
        cc = plsc.cumsum(crow)
        addr = jnp.full((NLANE,), r, jnp.int32)
        plsc.store_scatter(rpart, [addr], cs, mask=m15)
        plsc.store_scatter(rpart, [addr + NSEG_PAD], cc, mask=m15)

    pltpu.sync_copy(rpart, shared.at[sid])
    plsc.subcore_barrier()

    @pl.when(sid == 0)
    def _():
        for t in range(1, NS):
            pltpu.sync_copy(shared.at[t], tbuf)

            def add_body(j, _):
                sl = pl.ds(j * NLANE, NLANE)
                rpart[sl] = rpart[sl] + tbuf[sl]
                return 0

            lax.fori_loop(0, 2 * NSEG_PAD // NLANE, add_body, 0)
        pltpu.sync_copy(rpart, out_hbm.at[cid])


_segment_partials = functools.partial(
    pl.kernel,
    out_type=jax.ShapeDtypeStruct((NC, 2 * NSEG_PAD), jnp.float32),
    mesh=plsc.VectorSubcoreMesh(core_axis_name="c", subcore_axis_name="s"),
    compiler_params=pltpu.CompilerParams(needs_layout_passes=False),
    scratch_types=[
        pltpu.VMEM((CHUNK,), jnp.float32),
        pltpu.VMEM((CHUNK,), jnp.float32),
        pltpu.VMEM((CHUNK,), jnp.int32),
        pltpu.VMEM((CHUNK,), jnp.int32),
        pltpu.VMEM((NSEG_PAD * NLANE,), jnp.float32),
        pltpu.VMEM((NSEG_PAD * NLANE,), jnp.float32),
        pltpu.VMEM((2 * NSEG_PAD,), jnp.float32),
        pltpu.VMEM((2 * NSEG_PAD,), jnp.float32),
        pltpu.VMEM_SHARED((NS, 2 * NSEG_PAD), jnp.float32),
        pltpu.SemaphoreType.DMA,
        pltpu.SemaphoreType.DMA,
        pltpu.SemaphoreType.DMA,
        pltpu.SemaphoreType.DMA,
    ],
)(_sc_body)


def _finalize_body(s_ref, c_ref, t_ref, o_ref):
    s = s_ref[0] + s_ref[1]
    c = c_ref[0] + c_ref[1]
    seg = (lax.broadcasted_iota(jnp.int32, (8, 128), 0) * 128
           + lax.broadcasted_iota(jnp.int32, (8, 128), 1))
    mean = s / c
    d = jnp.where(seg < NSEG, mean - t_ref[...], 0.0)
    loss = jnp.sum(d * d) * jnp.float32(0.01 / NSEG)
    o_ref[...] = jnp.full((8, 128), loss, jnp.float32)


def kernel(x, idx, target_mean_weights):
    partials = _segment_partials(x, idx)
    p = partials.reshape(NC, 2, NSEG_PAD)
    sums = p[:, 0].reshape(NC, 8, 128)
    counts = p[:, 1].reshape(NC, 8, 128)
    t = jnp.pad(target_mean_weights, (0, NSEG_PAD - NSEG)).reshape(8, 128)
    out = pl.pallas_call(
        _finalize_body,
        out_shape=jax.ShapeDtypeStruct((8, 128), jnp.float32),
    )(sums, counts, t)
    return out[0, 0]


# speedup vs baseline: 221.9811x; 1.1775x over previous
import functools

import jax
import jax.numpy as jnp
from jax import lax
from jax.experimental import pallas as pl
from jax.experimental.pallas import tpu as pltpu
from jax.experimental.pallas import tpu_sc as plsc

N_EDGES_C = 6_400_000
NSEG = 1000
NSEG_PAD = 1024
NC = 2
NS = 16
NLANE = 16
NW = NC * NS
EDGES_PER_W = N_EDGES_C // NW
CHUNK = 20_000
NCHUNK = EDGES_PER_W // CHUNK
VPC = CHUNK // NLANE


def _sc_body(x_hbm, idx_hbm, out_hbm, xbuf0, xbuf1, ibuf0, ibuf1, acc_s,
             acc_c, rpart, tbuf, shared, semx0, semx1, semi0, semi1):
    cid = lax.axis_index("c")
    sid = lax.axis_index("s")
    wid = cid * NS + sid

    lane = lax.iota(jnp.int32, NLANE)
    ones = jnp.full((NLANE,), 1.0, jnp.float32)
    zeros = jnp.zeros((NLANE,), jnp.float32)
    semx = [semx0, semx1]
    semi = [semi0, semi1]
    xbufs = [xbuf0, xbuf1]
    ibufs = [ibuf0, ibuf1]

    base = wid * EDGES_PER_W

    def issue(ch, b):
        off = base + ch * CHUNK
        hx = pltpu.async_copy(x_hbm.at[pl.ds(off, CHUNK)], xbufs[b], semx[b])
        hi = pltpu.async_copy(idx_hbm.at[pl.ds(off, CHUNK)], ibufs[b],
                              semi[b])
        return hx, hi

    pending = [issue(0, 0), None]

    @plsc.parallel_loop(0, NSEG_PAD, unroll=8)
    def _(i):
        acc_s[pl.ds(i * NLANE, NLANE)] = zeros
        acc_c[pl.ds(i * NLANE, NLANE)] = zeros

    for ch in range(NCHUNK):
        b = ch & 1
        hx, hi = pending[b]
        hx.wait()
        hi.wait()
        if ch + 1 < NCHUNK:
            pending[1 - b] = issue(ch + 1, 1 - b)
        ib = ibufs[b]
        xb = xbufs[b]

        @plsc.parallel_loop(0, VPC, unroll=16)
        def _(i):
            o = i * NLANE
            iv = ib[pl.ds(o, NLANE)]
            xv = xb[pl.ds(o, NLANE)]
            addr = iv * NLANE + lane
            plsc.addupdate_scatter(acc_s, [addr], xv)
            plsc.addupdate_scatter(acc_c, [addr], ones)

    m15 = lane == 15

    @plsc.parallel_loop(0, NSEG_PAD, unroll=8)
    def _(r):
        srow = acc_s[pl.ds(r * NLANE, NLANE)]
        crow = acc_c[pl.ds(r * NLANE, NLANE)]
        cs = plsc.cumsum(srow)
        cc = plsc.cumsum(crow)
        addr = jnp.full((NLANE,), r, jnp.int32)
        plsc.store_scatter(rpart, [addr], cs, mask=m15)
        plsc.store_scatter(rpart, [addr + NSEG_PAD], cc, mask=m15)

    pltpu.sync_copy(rpart, shared.at[sid])
    plsc.subcore_barrier()

    sl0 = sid * 128
    handles = [
        pltpu.async_copy(shared.at[j, pl.ds(sl0, 128)],
                         tbuf.at[pl.ds(j * 128, 128)], semx0)
        for j in range(NS)
    ]
    for h in handles:
        h.wait()
    for k in range(8):
        tot = tbuf[pl.ds(k * NLANE, NLANE)]
        for j in range(1, NS):
            tot = tot + tbuf[pl.ds(j * 128 + k * NLANE, NLANE)]
        rpart[pl.ds(k * NLANE, NLANE)] = tot
    pltpu.sync_copy(rpart.at[pl.ds(0, 128)],
                    out_hbm.at[cid, pl.ds(sl0, 128)])


_segment_partials = functools.partial(
    pl.kernel,
    out_type=jax.ShapeDtypeStruct((NC, 2 * NSEG_PAD), jnp.float32),
    mesh=plsc.VectorSubcoreMesh(core_axis_name="c", subcore_axis_name="s"),
    compiler_params=pltpu.CompilerParams(needs_layout_passes=False),
    scratch_types=[
        pltpu.VMEM((CHUNK,), jnp.float32),
        pltpu.VMEM((CHUNK,), jnp.float32),
        pltpu.VMEM((CHUNK,), jnp.int32),
        pltpu.VMEM((CHUNK,), jnp.int32),
        pltpu.VMEM((NSEG_PAD * NLANE,), jnp.float32),
        pltpu.VMEM((NSEG_PAD * NLANE,), jnp.float32),
        pltpu.VMEM((2 * NSEG_PAD,), jnp.float32),
        pltpu.VMEM((2 * NSEG_PAD,), jnp.float32),
        pltpu.VMEM_SHARED((NS, 2 * NSEG_PAD), jnp.float32),
        pltpu.SemaphoreType.DMA,
        pltpu.SemaphoreType.DMA,
        pltpu.SemaphoreType.DMA,
        pltpu.SemaphoreType.DMA,
    ],
)(_sc_body)


def _finalize_body(s_ref, c_ref, t_ref, o_ref):
    s = s_ref[0] + s_ref[1]
    c = c_ref[0] + c_ref[1]
    seg = (lax.broadcasted_iota(jnp.int32, (8, 128), 0) * 128
           + lax.broadcasted_iota(jnp.int32, (8, 128), 1))
    mean = s / c
    d = jnp.where(seg < NSEG, mean - t_ref[...], 0.0)
    loss = jnp.sum(d * d) * jnp.float32(0.01 / NSEG)
    o_ref[...] = jnp.full((8, 128), loss, jnp.float32)


def kernel(x, idx, target_mean_weights):
    partials = _segment_partials(x, idx)
    p = partials.reshape(NC, 2, NSEG_PAD)
    sums = p[:, 0].reshape(NC, 8, 128)
    counts = p[:, 1].reshape(NC, 8, 128)
    t = jnp.pad(target_mean_weights, (0, NSEG_PAD - NSEG)).reshape(8, 128)
    out = pl.pallas_call(
        _finalize_body,
        out_shape=jax.ShapeDtypeStruct((8, 128), jnp.float32),
    )(sums, counts, t)
    return out[0, 0]


# speedup vs baseline: 241.2205x; 1.0867x over previous
import functools

import jax
import jax.numpy as jnp
from jax import lax
from jax.experimental import pallas as pl
from jax.experimental.pallas import tpu as pltpu
from jax.experimental.pallas import tpu_sc as plsc

N_EDGES_C = 6_400_000
NSEG = 1000
NSEG_PAD = 1024
NC = 2
NS = 16
NLANE = 16
NW = NC * NS
EDGES_PER_W = N_EDGES_C // NW
CHUNK = 20_000
NCHUNK = EDGES_PER_W // CHUNK
VPC = CHUNK // NLANE


def _sc_body(x_hbm, idx_hbm, out_hbm, xbuf0, xbuf1, ibuf0, ibuf1, acc_s,
             rpart, tbuf, shared, semx0, semx1, semi0, semi1):
    cid = lax.axis_index("c")
    sid = lax.axis_index("s")
    wid = cid * NS + sid

    lane = lax.iota(jnp.int32, NLANE)
    kbias = jnp.full((NLANE,), 4096.0, jnp.float32)
    zeros = jnp.zeros((NLANE,), jnp.float32)
    semx = [semx0, semx1]
    semi = [semi0, semi1]
    xbufs = [xbuf0, xbuf1]
    ibufs = [ibuf0, ibuf1]

    base = wid * EDGES_PER_W

    def issue(ch, b):
        off = base + ch * CHUNK
        hx = pltpu.async_copy(x_hbm.at[pl.ds(off, CHUNK)], xbufs[b], semx[b])
        hi = pltpu.async_copy(idx_hbm.at[pl.ds(off, CHUNK)], ibufs[b],
                              semi[b])
        return hx, hi

    pending = [issue(0, 0), None]

    @plsc.parallel_loop(0, NSEG_PAD, unroll=8)
    def _(i):
        acc_s[pl.ds(i * NLANE, NLANE)] = zeros

    for ch in range(NCHUNK):
        b = ch & 1
        hx, hi = pending[b]
        hx.wait()
        hi.wait()
        if ch + 1 < NCHUNK:
            pending[1 - b] = issue(ch + 1, 1 - b)
        ib = ibufs[b]
        xb = xbufs[b]

        @plsc.parallel_loop(0, VPC, unroll=16)
        def _(i):
            o = i * NLANE
            iv = ib[pl.ds(o, NLANE)]
            xv = xb[pl.ds(o, NLANE)]
            addr = iv * NLANE + lane
            plsc.addupdate_scatter(acc_s, [addr], xv + kbias)

    m15 = lane == 15

    @plsc.parallel_loop(0, NSEG_PAD, unroll=8)
    def _(r):
        srow = acc_s[pl.ds(r * NLANE, NLANE)]
        cs = plsc.cumsum(srow)
        cnt = (cs * (1.0 / 4096.0) + 0.5).astype(jnp.int32).astype(jnp.float32)
        sm = cs - cnt * 4096.0
        addr = jnp.full((NLANE,), r, jnp.int32)
        plsc.store_scatter(rpart, [addr], sm, mask=m15)
        plsc.store_scatter(rpart, [addr + NSEG_PAD], cnt, mask=m15)

    pltpu.sync_copy(rpart, shared.at[sid])
    plsc.subcore_barrier()

    sl0 = sid * 128
    handles = [
        pltpu.async_copy(shared.at[j, pl.ds(sl0, 128)],
                         tbuf.at[pl.ds(j * 128, 128)], semx0)
        for j in range(NS)
    ]
    for h in handles:
        h.wait()
    for k in range(8):
        tot = tbuf[pl.ds(k * NLANE, NLANE)]
        for j in range(1, NS):
            tot = tot + tbuf[pl.ds(j * 128 + k * NLANE, NLANE)]
        rpart[pl.ds(k * NLANE, NLANE)] = tot
    pltpu.sync_copy(rpart.at[pl.ds(0, 128)],
                    out_hbm.at[cid, pl.ds(sl0, 128)])


_segment_partials = functools.partial(
    pl.kernel,
    out_type=jax.ShapeDtypeStruct((NC, 2 * NSEG_PAD), jnp.float32),
    mesh=plsc.VectorSubcoreMesh(core_axis_name="c", subcore_axis_name="s"),
    compiler_params=pltpu.CompilerParams(needs_layout_passes=False),
    scratch_types=[
        pltpu.VMEM((CHUNK,), jnp.float32),
        pltpu.VMEM((CHUNK,), jnp.float32),
        pltpu.VMEM((CHUNK,), jnp.int32),
        pltpu.VMEM((CHUNK,), jnp.int32),
        pltpu.VMEM((NSEG_PAD * NLANE,), jnp.float32),
        pltpu.VMEM((2 * NSEG_PAD,), jnp.float32),
        pltpu.VMEM((2 * NSEG_PAD,), jnp.float32),
        pltpu.VMEM_SHARED((NS, 2 * NSEG_PAD), jnp.float32),
        pltpu.SemaphoreType.DMA,
        pltpu.SemaphoreType.DMA,
        pltpu.SemaphoreType.DMA,
        pltpu.SemaphoreType.DMA,
    ],
)(_sc_body)


def _finalize_body(s_ref, c_ref, t_ref, o_ref):
    s = s_ref[0] + s_ref[1]
    c = c_ref[0] + c_ref[1]
    seg = (lax.broadcasted_iota(jnp.int32, (8, 128), 0) * 128
           + lax.broadcasted_iota(jnp.int32, (8, 128), 1))
    mean = s / c
    d = jnp.where(seg < NSEG, mean - t_ref[...], 0.0)
    loss = jnp.sum(d * d) * jnp.float32(0.01 / NSEG)
    o_ref[...] = jnp.full((8, 128), loss, jnp.float32)


def kernel(x, idx, target_mean_weights):
    partials = _segment_partials(x, idx)
    p = partials.reshape(NC, 2, NSEG_PAD)
    sums = p[:, 0].reshape(NC, 8, 128)
    counts = p[:, 1].reshape(NC, 8, 128)
    t = jnp.pad(target_mean_weights, (0, NSEG_PAD - NSEG)).reshape(8, 128)
    out = pl.pallas_call(
        _finalize_body,
        out_shape=jax.ShapeDtypeStruct((8, 128), jnp.float32),
    )(sums, counts, t)
    return out[0, 0]


# speedup vs baseline: 250.9266x; 1.0402x over previous
import functools

import jax
import jax.numpy as jnp
from jax import lax
from jax.experimental import pallas as pl
from jax.experimental.pallas import tpu as pltpu
from jax.experimental.pallas import tpu_sc as plsc

N_EDGES_C = 6_400_000
NSEG = 1000
NSEG_PAD = 1024
NC = 2
NS = 16
NLANE = 16
NW = NC * NS
EDGES_PER_W = N_EDGES_C // NW
CHUNK = 20_000
NCHUNK = EDGES_PER_W // CHUNK
VPC = CHUNK // NLANE


def _sc_body(x_hbm, idx_hbm, out_hbm, xbuf0, xbuf1, ibuf0, ibuf1, acc_s,
             rpart, tbuf, shared, semx0, semx1, semi0, semi1):
    cid = lax.axis_index("c")
    sid = lax.axis_index("s")
    wid = cid * NS + sid

    lane = lax.iota(jnp.int32, NLANE)
    kbias = jnp.full((NLANE,), 4096.0, jnp.float32)
    zeros = jnp.zeros((NLANE,), jnp.float32)
    semx = [semx0, semx1]
    semi = [semi0, semi1]
    xbufs = [xbuf0, xbuf1]
    ibufs = [ibuf0, ibuf1]

    base = wid * EDGES_PER_W

    def issue(ch, b):
        off = base + ch * CHUNK
        hx = pltpu.async_copy(x_hbm.at[pl.ds(off, CHUNK)], xbufs[b], semx[b])
        hi = pltpu.async_copy(idx_hbm.at[pl.ds(off, CHUNK)], ibufs[b],
                              semi[b])
        return hx, hi

    pending = [issue(0, 0), None]

    @plsc.parallel_loop(0, NSEG_PAD, unroll=8)
    def _(i):
        acc_s[pl.ds(i * NLANE, NLANE)] = zeros

    for ch in range(NCHUNK):
        b = ch & 1
        hx, hi = pending[b]
        hx.wait()
        hi.wait()
        if ch + 1 < NCHUNK:
            pending[1 - b] = issue(ch + 1, 1 - b)
        ib = ibufs[b]
        xb = xbufs[b]

        @plsc.parallel_loop(0, VPC, unroll=16)
        def _(i):
            o = i * NLANE
            iv = ib[pl.ds(o, NLANE)]
            xv = xb[pl.ds(o, NLANE)]
            addr = iv * NLANE + lane
            plsc.addupdate_scatter(acc_s, [addr], xv + kbias)

    m15 = lane == 15

    @plsc.parallel_loop(0, NSEG_PAD, unroll=8)
    def _(r):
        srow = acc_s[pl.ds(r * NLANE, NLANE)]
        cs = plsc.cumsum(srow)
        cnt = (cs * (1.0 / 4096.0) + 0.5).astype(jnp.int32).astype(jnp.float32)
        sm = cs - cnt * 4096.0
        addr = jnp.full((NLANE,), r, jnp.int32)
        plsc.store_scatter(rpart, [addr], sm, mask=m15)
        plsc.store_scatter(rpart, [addr + NSEG_PAD], cnt, mask=m15)

    pltpu.sync_copy(rpart, shared.at[sid])
    plsc.subcore_barrier()

    sl0 = sid * 128
    handles = [
        pltpu.async_copy(shared.at[j, pl.ds(sl0, 128)],
                         tbuf.at[pl.ds(j * 128, 128)], semx0)
        for j in range(NS)
    ]
    for h in handles:
        h.wait()
    for k in range(8):
        tot = tbuf[pl.ds(k * NLANE, NLANE)]
        for j in range(1, NS):
            tot = tot + tbuf[pl.ds(j * 128 + k * NLANE, NLANE)]
        rpart[pl.ds(k * NLANE, NLANE)] = tot
    pltpu.sync_copy(rpart.at[pl.ds(0, 128)],
                    out_hbm.at[cid, pl.ds(sl0, 128)])


_segment_partials = functools.partial(
    pl.kernel,
    out_type=jax.ShapeDtypeStruct((NC, 2 * NSEG_PAD), jnp.float32),
    mesh=plsc.VectorSubcoreMesh(core_axis_name="c", subcore_axis_name="s"),
    compiler_params=pltpu.CompilerParams(needs_layout_passes=False),
    scratch_types=[
        pltpu.VMEM((CHUNK,), jnp.float32),
        pltpu.VMEM((CHUNK,), jnp.float32),
        pltpu.VMEM((CHUNK,), jnp.int32),
        pltpu.VMEM((CHUNK,), jnp.int32),
        pltpu.VMEM((NSEG_PAD * NLANE,), jnp.float32),
        pltpu.VMEM((2 * NSEG_PAD,), jnp.float32),
        pltpu.VMEM((2 * NSEG_PAD,), jnp.float32),
        pltpu.VMEM_SHARED((NS, 2 * NSEG_PAD), jnp.float32),
        pltpu.SemaphoreType.DMA,
        pltpu.SemaphoreType.DMA,
        pltpu.SemaphoreType.DMA,
        pltpu.SemaphoreType.DMA,
    ],
)(_sc_body)


def _finalize_body(p_ref, t_ref, o_ref):
    p = p_ref[...].reshape(NC, 2, 8, 128)
    s = p[0, 0] + p[1, 0]
    c = p[0, 1] + p[1, 1]
    seg = (lax.broadcasted_iota(jnp.int32, (8, 128), 0) * 128
           + lax.broadcasted_iota(jnp.int32, (8, 128), 1))
    mean = s / c
    d = jnp.where(seg < NSEG, mean - t_ref[...], 0.0)
    loss = jnp.sum(d * d) * jnp.float32(0.01 / NSEG)
    o_ref[...] = jnp.full((8, 128), loss, jnp.float32)


def kernel(x, idx, target_mean_weights):
    partials = _segment_partials(x, idx)
    t = jnp.pad(target_mean_weights, (0, NSEG_PAD - NSEG)).reshape(8, 128)
    out = pl.pallas_call(
        _finalize_body,
        out_shape=jax.ShapeDtypeStruct((8, 128), jnp.float32),
    )(partials, t)
    return out[0, 0]


# speedup vs baseline: 257.0194x; 1.0243x over previous
import functools

import jax
import jax.numpy as jnp
from jax import lax
from jax.experimental import pallas as pl
from jax.experimental.pallas import tpu as pltpu
from jax.experimental.pallas import tpu_sc as plsc

N_EDGES_C = 6_400_000
NSEG = 1000
NSEG_PAD = 1024
NC = 2
NS = 16
NLANE = 16
NW = NC * NS
EDGES_PER_W = N_EDGES_C // NW
CHUNK = 20_000
NCHUNK = EDGES_PER_W // CHUNK
VPC = CHUNK // NLANE


def _sc_body(x_hbm, idx_hbm, out_hbm, xbuf0, xbuf1, ibuf0, ibuf1, acc_s,
             rpart, tbuf, shared, semx0, semx1, semi0, semi1):
    cid = lax.axis_index("c")
    sid = lax.axis_index("s")
    wid = cid * NS + sid

    lane = lax.iota(jnp.int32, NLANE)
    kbias = jnp.full((NLANE,), 4096.0, jnp.float32)
    zeros = jnp.zeros((NLANE,), jnp.float32)
    semx = [semx0, semx1]
    semi = [semi0, semi1]
    xbufs = [xbuf0, xbuf1]
    ibufs = [ibuf0, ibuf1]

    base = wid * EDGES_PER_W

    def issue(ch, b):
        off = base + ch * CHUNK
        hx = pltpu.async_copy(x_hbm.at[pl.ds(off, CHUNK)], xbufs[b], semx[b])
        hi = pltpu.async_copy(idx_hbm.at[pl.ds(off, CHUNK)], ibufs[b],
                              semi[b])
        return hx, hi

    pending = [issue(0, 0), None]

    @plsc.parallel_loop(0, NSEG_PAD, unroll=8)
    def _(i):
        acc_s[pl.ds(i * NLANE, NLANE)] = zeros

    for ch in range(NCHUNK):
        b = ch & 1
        hx, hi = pending[b]
        hx.wait()
        hi.wait()
        if ch + 1 < NCHUNK:
            pending[1 - b] = issue(ch + 1, 1 - b)
        ib = ibufs[b]
        xb = xbufs[b]

        @plsc.parallel_loop(0, VPC, unroll=16)
        def _(i):
            o = i * NLANE
            iv = ib[pl.ds(o, NLANE)]
            xv = xb[pl.ds(o, NLANE)]
            addr = iv * NLANE + lane
            plsc.addupdate_scatter(acc_s, [addr], xv + kbias)

    m15 = lane == 15

    @plsc.parallel_loop(0, NSEG_PAD, unroll=8)
    def _(r):
        srow = acc_s[pl.ds(r * NLANE, NLANE)]
        cs = plsc.cumsum(srow)
        cnt = (cs * (1.0 / 4096.0) + 0.5).astype(jnp.int32).astype(jnp.float32)
        sm = cs - cnt * 4096.0
        addr = jnp.full((NLANE,), r, jnp.int32)
        plsc.store_scatter(rpart, [addr], sm, mask=m15)
        plsc.store_scatter(rpart, [addr + NSEG_PAD], cnt, mask=m15)

    pltpu.sync_copy(rpart, shared.at[sid])
    plsc.subcore_barrier()

    sl0 = sid * 128
    handles = [
        pltpu.async_copy(shared.at[j, pl.ds(sl0, 128)],
                         tbuf.at[pl.ds(j * 128, 128)], semx0)
        for j in range(NS)
    ]
    for h in handles:
        h.wait()
    for k in range(8):
        tot = tbuf[pl.ds(k * NLANE, NLANE)]
        for j in range(1, NS):
            tot = tot + tbuf[pl.ds(j * 128 + k * NLANE, NLANE)]
        rpart[pl.ds(k * NLANE, NLANE)] = tot
    pltpu.sync_copy(rpart.at[pl.ds(0, 128)],
                    out_hbm.at[cid, pl.ds(sl0, 128)])


_segment_partials = functools.partial(
    pl.kernel,
    out_type=jax.ShapeDtypeStruct((NC, 2 * NSEG_PAD), jnp.float32),
    mesh=plsc.VectorSubcoreMesh(core_axis_name="c", subcore_axis_name="s"),
    compiler_params=pltpu.CompilerParams(needs_layout_passes=False),
    scratch_types=[
        pltpu.VMEM((CHUNK,), jnp.float32),
        pltpu.VMEM((CHUNK,), jnp.float32),
        pltpu.VMEM((CHUNK,), jnp.int32),
        pltpu.VMEM((CHUNK,), jnp.int32),
        pltpu.VMEM((NSEG_PAD * NLANE,), jnp.float32),
        pltpu.VMEM((2 * NSEG_PAD,), jnp.float32),
        pltpu.VMEM((2 * NSEG_PAD,), jnp.float32),
        pltpu.VMEM_SHARED((NS, 2 * NSEG_PAD), jnp.float32),
        pltpu.SemaphoreType.DMA,
        pltpu.SemaphoreType.DMA,
        pltpu.SemaphoreType.DMA,
        pltpu.SemaphoreType.DMA,
    ],
)(_sc_body)


def _finalize_body(p_ref, t_ref, o_ref):
    p = p_ref[...].reshape(NC, 2, 8, 128)
    s = p[0, 0] + p[1, 0]
    c = p[0, 1] + p[1, 1]
    seg = (lax.broadcasted_iota(jnp.int32, (8, 128), 0) * 128
           + lax.broadcasted_iota(jnp.int32, (8, 128), 1))
    mean = s / c
    t = t_ref[...].reshape(8, 128)
    d = jnp.where(seg < NSEG, mean - t, 0.0)
    loss = jnp.sum(d * d) * jnp.float32(0.01 / NSEG)
    o_ref[...] = jnp.full((1, 1), loss, jnp.float32)


def kernel(x, idx, target_mean_weights):
    partials = _segment_partials(x, idx)
    t = jnp.pad(target_mean_weights, (0, NSEG_PAD - NSEG))
    out = pl.pallas_call(
        _finalize_body,
        out_shape=jax.ShapeDtypeStruct((1, 1), jnp.float32),
    )(partials, t)
    return out[0, 0]
